# Initial kernel scaffold; baseline (speedup 1.0000x reference)
import jax
import jax.numpy as jnp
from jax.experimental import pallas as pl


def kernel(img, params):
    raise NotImplementedError("write your pallas kernel here")


# speedup vs baseline: 552.3196x; 552.3196x over previous
import functools

import jax
import jax.numpy as jnp
from jax import lax
from jax.experimental import pallas as pl
from jax.experimental.pallas import tpu as pltpu
from jax.experimental.pallas import tpu_sc as plsc

N_CTRL = 17
LANES = 16

_GATHER_DNUMS = lax.GatherDimensionNumbers(
    offset_dims=(), collapsed_slice_dims=(0,), start_index_map=(0,)
)


def _vreg_gather(table, idx):
    return lax.gather(
        table,
        idx[:, None],
        _GATHER_DNUMS,
        slice_sizes=(1,),
        mode=lax.GatherScatterMode.PROMISE_IN_BOUNDS,
    )


def _tone_curve_sc(flat, a_tab, d_tab, n_planes, plane_size):
    n_workers = 32
    planes_per_w = n_planes // n_workers
    chunk = 16384
    chunks_per_plane = plane_size // chunk
    mesh = plsc.VectorSubcoreMesh(core_axis_name="c", subcore_axis_name="s")

    @functools.partial(
        pl.kernel,
        mesh=mesh,
        out_type=jax.ShapeDtypeStruct((n_planes * plane_size,), jnp.float32),
        scratch_types=[
            pltpu.VMEM((chunk,), jnp.float32),
            pltpu.VMEM((LANES,), jnp.float32),
            pltpu.VMEM((LANES,), jnp.float32),
        ],
    )
    def body(flat_hbm, a_hbm, d_hbm, out_hbm, buf, a_v, d_v):
        wid = lax.axis_index("s") * 2 + lax.axis_index("c")
        for p in range(planes_per_w):
            plane = wid * planes_per_w + p
            pltpu.sync_copy(a_hbm.at[plane], a_v)
            pltpu.sync_copy(d_hbm.at[plane], d_v)
            a_reg = a_v[...]
            d_reg = d_v[...]
            plane_base = plane * plane_size

            def chunk_body(g, _, plane_base=plane_base, a_reg=a_reg, d_reg=d_reg):
                base = plane_base + g * chunk
                pltpu.sync_copy(flat_hbm.at[pl.ds(base, chunk)], buf)

                def vec_body(i, _):
                    x = buf[pl.ds(i * LANES, LANES)]
                    scaled = x * jnp.float32(N_CTRL - 1)
                    lo = lax.convert_element_type(scaled, jnp.int32)
                    lo = lax.min(lax.max(lo, 0), N_CTRL - 2)
                    frac = scaled - lax.convert_element_type(lo, jnp.float32)
                    av = _vreg_gather(a_reg, lo)
                    dv = _vreg_gather(d_reg, lo)
                    buf[pl.ds(i * LANES, LANES)] = av + dv * frac
                    return 0

                lax.fori_loop(0, chunk // LANES, vec_body, 0)
                pltpu.sync_copy(buf, out_hbm.at[pl.ds(base, chunk)])
                return 0

            lax.fori_loop(0, chunks_per_plane, chunk_body, 0)

    return body(flat, a_tab, d_tab)


def kernel(img, params):
    B, C, H, W = img.shape
    K = N_CTRL
    offsets = params.reshape(B, C, K)
    identity_y = jnp.linspace(0.0, 1.0, K, dtype=jnp.float32)
    cy = jnp.clip(identity_y[None, None, :] + offsets, 0.0, 1.0)
    a_tab = cy[..., : K - 1].reshape(B * C, K - 1)
    d_tab = (cy[..., 1:] - cy[..., : K - 1]).reshape(B * C, K - 1)
    flat = img.reshape(B * C * H * W)
    out = _tone_curve_sc(flat, a_tab, d_tab, B * C, H * W)
    return out.reshape(B, C, H, W)


# speedup vs baseline: 629.4775x; 1.1397x over previous
import functools

import jax
import jax.numpy as jnp
from jax import lax
from jax.experimental import pallas as pl
from jax.experimental.pallas import tpu as pltpu
from jax.experimental.pallas import tpu_sc as plsc

N_CTRL = 17
LANES = 16

_GATHER_DNUMS = lax.GatherDimensionNumbers(
    offset_dims=(), collapsed_slice_dims=(0,), start_index_map=(0,)
)


def _vreg_gather(table, idx):
    return lax.gather(
        table,
        idx[:, None],
        _GATHER_DNUMS,
        slice_sizes=(1,),
        mode=lax.GatherScatterMode.PROMISE_IN_BOUNDS,
    )


def _tone_curve_sc(flat, a_tab, d_tab, n_planes, plane_size):
    n_workers = 32
    planes_per_w = n_planes // n_workers
    chunk = 16384
    chunks_per_plane = plane_size // chunk
    n_chunks = planes_per_w * chunks_per_plane
    mesh = plsc.VectorSubcoreMesh(core_axis_name="c", subcore_axis_name="s")

    @functools.partial(
        pl.kernel,
        mesh=mesh,
        out_type=jax.ShapeDtypeStruct((n_planes * plane_size,), jnp.float32),
        scratch_types=[
            pltpu.VMEM((planes_per_w * LANES,), jnp.float32),
            pltpu.VMEM((planes_per_w * LANES,), jnp.float32),
            pltpu.VMEM((chunk,), jnp.float32),
            pltpu.VMEM((chunk,), jnp.float32),
            pltpu.VMEM((chunk,), jnp.float32),
            pltpu.VMEM((chunk,), jnp.float32),
            pltpu.SemaphoreType.DMA,
            pltpu.SemaphoreType.DMA,
            pltpu.SemaphoreType.DMA,
            pltpu.SemaphoreType.DMA,
        ],
    )
    def body(flat_hbm, a_hbm, d_hbm, out_hbm, lut_a, lut_d,
             in0, in1, ob0, ob1, si0, si1, so0, so1):
        wid = lax.axis_index("s") * 2 + lax.axis_index("c")
        lut_base = wid * planes_per_w * LANES
        pltpu.sync_copy(a_hbm.at[pl.ds(lut_base, planes_per_w * LANES)], lut_a)
        pltpu.sync_copy(d_hbm.at[pl.ds(lut_base, planes_per_w * LANES)], lut_d)
        wbase = wid * planes_per_w * plane_size

        def in_copy(c, buf, sem):
            return pltpu.make_async_copy(
                flat_hbm.at[pl.ds(wbase + c * chunk, chunk)], buf, sem)

        def out_copy(c, buf, sem):
            return pltpu.make_async_copy(
                buf, out_hbm.at[pl.ds(wbase + c * chunk, chunk)], sem)

        bufs = ((in0, si0, ob0, so0), (in1, si1, ob1, so1))

        in_copy(0, in0, si0).start()

        def process(c, bi):
            ibuf, isem, obuf, osem = bufs[bi]
            n_ibuf, n_isem = bufs[1 - bi][0], bufs[1 - bi][1]

            @pl.when(c + 1 < n_chunks)
            def _():
                in_copy(c + 1, n_ibuf, n_isem).start()

            in_copy(c, ibuf, isem).wait()

            @pl.when(c >= 2)
            def _():
                out_copy(c - 2, obuf, osem).wait()

            p_idx = c // chunks_per_plane
            a_reg = lut_a[pl.ds(p_idx * LANES, LANES)]
            d_reg = lut_d[pl.ds(p_idx * LANES, LANES)]

            def vec_body(i, _):
                x = ibuf[pl.ds(i * LANES, LANES)]
                scaled = x * jnp.float32(N_CTRL - 1)
                lo = lax.convert_element_type(scaled, jnp.int32)
                lo = lax.min(lax.max(lo, 0), N_CTRL - 2)
                frac = scaled - lax.convert_element_type(lo, jnp.float32)
                av = _vreg_gather(a_reg, lo)
                dv = _vreg_gather(d_reg, lo)
                obuf[pl.ds(i * LANES, LANES)] = av + dv * frac
                return 0

            lax.fori_loop(0, chunk // LANES, vec_body, 0, unroll=8)
            out_copy(c, obuf, osem).start()

        def pair_body(gp, _):
            process(gp * 2, 0)
            process(gp * 2 + 1, 1)
            return 0

        lax.fori_loop(0, n_chunks // 2, pair_body, 0)
        out_copy(n_chunks - 2, ob0, so0).wait()
        out_copy(n_chunks - 1, ob1, so1).wait()

    return body(flat, a_tab, d_tab)


def kernel(img, params):
    B, C, H, W = img.shape
    K = N_CTRL
    offsets = params.reshape(B, C, K)
    identity_y = jnp.linspace(0.0, 1.0, K, dtype=jnp.float32)
    cy = jnp.clip(identity_y[None, None, :] + offsets, 0.0, 1.0)
    a_tab = cy[..., : K - 1].reshape(B * C * (K - 1))
    d_tab = (cy[..., 1:] - cy[..., : K - 1]).reshape(B * C * (K - 1))
    flat = img.reshape(B * C * H * W)
    out = _tone_curve_sc(flat, a_tab, d_tab, B * C, H * W)
    return out.reshape(B, C, H, W)


# speedup vs baseline: 1720.7306x; 2.7336x over previous
import functools

import jax
import jax.numpy as jnp
from jax import lax
from jax.experimental import pallas as pl
from jax.experimental.pallas import tpu as pltpu
from jax.experimental.pallas import tpu_sc as plsc

N_CTRL = 17
LANES = 16

_GATHER_DNUMS = lax.GatherDimensionNumbers(
    offset_dims=(), collapsed_slice_dims=(0,), start_index_map=(0,)
)


def _vreg_gather(table, idx):
    return lax.gather(
        table,
        idx[:, None],
        _GATHER_DNUMS,
        slice_sizes=(1,),
        mode=lax.GatherScatterMode.PROMISE_IN_BOUNDS,
    )


def _tone_curve_sc(flat, a_tab, d_tab, n_planes, plane_size):
    n_workers = 32
    planes_per_w = n_planes // n_workers
    chunk = 16384
    chunks_per_plane = plane_size // chunk
    n_chunks = planes_per_w * chunks_per_plane
    mesh = plsc.VectorSubcoreMesh(core_axis_name="c", subcore_axis_name="s")

    @functools.partial(
        pl.kernel,
        mesh=mesh,
        out_type=jax.ShapeDtypeStruct((n_planes * plane_size,), jnp.float32),
        scratch_types=[
            pltpu.VMEM((planes_per_w * LANES,), jnp.float32),
            pltpu.VMEM((planes_per_w * LANES,), jnp.float32),
            pltpu.VMEM((chunk,), jnp.float32),
            pltpu.VMEM((chunk,), jnp.float32),
            pltpu.VMEM((chunk,), jnp.float32),
            pltpu.VMEM((chunk,), jnp.float32),
            pltpu.SemaphoreType.DMA,
            pltpu.SemaphoreType.DMA,
            pltpu.SemaphoreType.DMA,
            pltpu.SemaphoreType.DMA,
        ],
    )
    def body(flat_hbm, a_hbm, d_hbm, out_hbm, lut_a, lut_d,
             in0, in1, ob0, ob1, si0, si1, so0, so1):
        wid = lax.axis_index("s") * 2 + lax.axis_index("c")
        lut_base = wid * planes_per_w * LANES
        pltpu.sync_copy(a_hbm.at[pl.ds(lut_base, planes_per_w * LANES)], lut_a)
        pltpu.sync_copy(d_hbm.at[pl.ds(lut_base, planes_per_w * LANES)], lut_d)
        wbase = wid * planes_per_w * plane_size

        def in_copy(c, buf, sem):
            return pltpu.make_async_copy(
                flat_hbm.at[pl.ds(wbase + c * chunk, chunk)], buf, sem)

        def out_copy(c, buf, sem):
            return pltpu.make_async_copy(
                buf, out_hbm.at[pl.ds(wbase + c * chunk, chunk)], sem)

        bufs = ((in0, si0, ob0, so0), (in1, si1, ob1, so1))

        in_copy(0, in0, si0).start()

        def process(c, bi):
            ibuf, isem, obuf, osem = bufs[bi]
            n_ibuf, n_isem = bufs[1 - bi][0], bufs[1 - bi][1]

            @pl.when(c + 1 < n_chunks)
            def _():
                in_copy(c + 1, n_ibuf, n_isem).start()

            in_copy(c, ibuf, isem).wait()

            @pl.when(c >= 2)
            def _():
                out_copy(c - 2, obuf, osem).wait()

            p_idx = c // chunks_per_plane
            a_reg = lut_a[pl.ds(p_idx * LANES, LANES)]
            d_reg = lut_d[pl.ds(p_idx * LANES, LANES)]

            @plsc.parallel_loop(0, chunk, step=LANES, unroll=8)
            def _(off):
                x = ibuf[pl.ds(off, LANES)]
                scaled = x * jnp.float32(N_CTRL - 1)
                lo = lax.convert_element_type(scaled, jnp.int32)
                lo = lax.min(lax.max(lo, 0), N_CTRL - 2)
                frac = scaled - lax.convert_element_type(lo, jnp.float32)
                av = _vreg_gather(a_reg, lo)
                dv = _vreg_gather(d_reg, lo)
                obuf[pl.ds(off, LANES)] = av + dv * frac
            out_copy(c, obuf, osem).start()

        def pair_body(gp, _):
            process(gp * 2, 0)
            process(gp * 2 + 1, 1)
            return 0

        lax.fori_loop(0, n_chunks // 2, pair_body, 0)
        out_copy(n_chunks - 2, ob0, so0).wait()
        out_copy(n_chunks - 1, ob1, so1).wait()

    return body(flat, a_tab, d_tab)


def kernel(img, params):
    B, C, H, W = img.shape
    K = N_CTRL
    offsets = params.reshape(B, C, K)
    identity_y = jnp.linspace(0.0, 1.0, K, dtype=jnp.float32)
    cy = jnp.clip(identity_y[None, None, :] + offsets, 0.0, 1.0)
    a_tab = cy[..., : K - 1].reshape(B * C * (K - 1))
    d_tab = (cy[..., 1:] - cy[..., : K - 1]).reshape(B * C * (K - 1))
    flat = img.reshape(B * C * H * W)
    out = _tone_curve_sc(flat, a_tab, d_tab, B * C, H * W)
    return out.reshape(B, C, H, W)
